# Initial kernel scaffold; baseline (speedup 1.0000x reference)
import jax
import jax.numpy as jnp
from jax.experimental import pallas as pl


def kernel(x, edge_index, batch, frag, frag_edge_index, frag_batch, target, params):
    raise NotImplementedError("write your pallas kernel here")


# speedup vs baseline: 1.4977x; 1.4977x over previous
import functools

import jax
import jax.numpy as jnp
from jax import lax
from jax.experimental import pallas as pl
from jax.experimental.pallas import tpu as pltpu
from jax.experimental.pallas import tpu_sc as plsc

NG = 256
HI = jax.lax.Precision.HIGHEST
BM = 512
EB = 128


def _pad_rows(a, n):
    return jnp.pad(a, ((0, n - a.shape[0]),) + ((0, 0),) * (a.ndim - 1))


@functools.partial(jax.jit, static_argnames=("np_", "ch", "nchunk", "d", "nb"))
def _sc_edge_agg(x, src, dst, *, np_, ch, nchunk, d, nb):
    ept = EB * nb
    cpt = ch // 16
    zr = 16
    mesh = plsc.VectorSubcoreMesh(core_axis_name="c", subcore_axis_name="s")

    @functools.partial(
        pl.kernel, mesh=mesh,
        out_type=jax.ShapeDtypeStruct((np_, d), jnp.float32),
        scratch_types=[
            pltpu.VMEM((EB,), jnp.int32),
            pltpu.VMEM((EB,), jnp.int32),
            pltpu.VMEM((EB,), jnp.int32),
            pltpu.VMEM((EB, d), jnp.float32),
            pltpu.VMEM((zr, d), jnp.float32),
            pltpu.VMEM_SHARED((ch + 16, d), jnp.float32),
            pltpu.SemaphoreType.DMA,
        ],
    )
    def k(x_h, src_h, dst_h, agg_h, sbuf, dbuf, lbuf, rows, zbuf, acc, sem):
        c = lax.axis_index("c")
        s = lax.axis_index("s")
        for r in range(zr):
            for q in range(d // 16):
                zbuf[r, pl.ds(q * 16, 16)] = jnp.zeros((16,), jnp.float32)
        for p in range(nchunk // 2):
            chunk = c + 2 * p
            base = chunk * ch
            for r in range(cpt // zr):
                pltpu.sync_copy(zbuf, acc.at[pl.ds(s * cpt + r * zr, zr)])
            plsc.subcore_barrier()

            def body(b, _):
                off = s * ept + b * EB
                pltpu.sync_copy(src_h.at[pl.ds(off, EB)], sbuf)
                pltpu.sync_copy(dst_h.at[pl.ds(off, EB)], dbuf)
                cp = pltpu.async_copy(x_h.at[sbuf], rows, sem)
                for j in range(EB // 16):
                    dv = dbuf[pl.ds(j * 16, 16)] - base
                    oob = (dv < 0) | (dv >= ch)
                    lbuf[pl.ds(j * 16, 16)] = jnp.where(oob, ch, dv)
                cp.wait()
                pltpu.sync_copy(rows, acc.at[lbuf], add=True)
                return 0

            lax.fori_loop(0, nb, body, 0)
            plsc.subcore_barrier()
            pltpu.sync_copy(acc.at[pl.ds(s * cpt, cpt)],
                            agg_h.at[pl.ds(base + s * cpt, cpt)])
            plsc.subcore_barrier()

    return k(x, src, dst)


@functools.partial(jax.jit, static_argnames=("np_",))
def _sc_embed_gather(table, idx, *, np_):
    rpt = np_ // 32
    nb = rpt // 16
    mesh = plsc.VectorSubcoreMesh(core_axis_name="c", subcore_axis_name="s")

    @functools.partial(
        pl.kernel, mesh=mesh,
        out_type=jax.ShapeDtypeStruct((np_, 128), jnp.float32),
        scratch_types=[
            pltpu.VMEM((16,), jnp.int32),
            pltpu.VMEM((16, 128), jnp.float32),
            pltpu.SemaphoreType.DMA,
        ],
    )
    def k(tab_h, idx_h, out_h, ibuf, rows, sem):
        c = lax.axis_index("c")
        s = lax.axis_index("s")
        w = s * 2 + c

        def body(b, _):
            off = w * rpt + b * 16
            pltpu.sync_copy(idx_h.at[pl.ds(off, 16)], ibuf)
            pltpu.async_copy(tab_h.at[ibuf], rows, sem).wait()
            pltpu.sync_copy(rows, out_h.at[pl.ds(off, 16)])
            return 0

        lax.fori_loop(0, nb, body, 0)

    return k(table, idx)


def _gin_mlp(x, agg, eps, W1, b1, W2, b2, *, nreal):
    np_, din = x.shape
    dh = W2.shape[1]
    grid = (np_ // BM,)

    def body(eps_ref, x_ref, a_ref, w1_ref, b1_ref, w2_ref, b2_ref,
             u_ref, t_ref):
        i = pl.program_id(0)
        h = x_ref[...] * (1.0 + eps_ref[0, 0]) + a_ref[...]
        t = jnp.maximum(jnp.dot(h, w1_ref[...],
                                preferred_element_type=jnp.float32)
                        + b1_ref[...], 0.0)
        u = jnp.dot(t, w2_ref[...],
                    preferred_element_type=jnp.float32) + b2_ref[...]
        rows = i * BM + lax.broadcasted_iota(jnp.int32, (BM, 1), 0)
        u_ref[...] = jnp.where(rows < nreal, u, 0.0)
        t_ref[...] = t

    return pl.pallas_call(
        body,
        grid=grid,
        in_specs=[
            pl.BlockSpec(memory_space=pltpu.SMEM),
            pl.BlockSpec((BM, din), lambda i: (i, 0)),
            pl.BlockSpec((BM, din), lambda i: (i, 0)),
            pl.BlockSpec((din, dh), lambda i: (0, 0)),
            pl.BlockSpec((1, dh), lambda i: (0, 0)),
            pl.BlockSpec((dh, dh), lambda i: (0, 0)),
            pl.BlockSpec((1, dh), lambda i: (0, 0)),
        ],
        out_specs=[pl.BlockSpec((BM, dh), lambda i: (i, 0)),
                   pl.BlockSpec((BM, dh), lambda i: (i, 0))],
        out_shape=[jax.ShapeDtypeStruct((np_, dh), jnp.float32),
                   jax.ShapeDtypeStruct((nreal, dh), jnp.float32)],
    )(jnp.reshape(eps, (1, 1)), x, agg, W1, jnp.reshape(b1, (1, -1)),
      W2, jnp.reshape(b2, (1, -1)))


def _bn_relu(u, m, v, g, be, *, nreal, pool_ids=None):
    np_, dh = u.shape
    grid = (np_ // BM,)
    with_pool = pool_ids is not None

    def body(*refs):
        if with_pool:
            u_ref, m_ref, v_ref, g_ref, be_ref, b_ref, xp_ref, p_ref = refs
        else:
            u_ref, m_ref, v_ref, g_ref, be_ref, xp_ref = refs
        i = pl.program_id(0)
        sq = jnp.sqrt(v_ref[...] + 1e-5)
        xp = jnp.maximum((u_ref[...] - m_ref[...]) / sq * g_ref[...]
                         + be_ref[...], 0.0)
        rows = i * BM + lax.broadcasted_iota(jnp.int32, (BM, 1), 0)
        xp = jnp.where(rows < nreal, xp, 0.0)
        xp_ref[...] = xp
        if with_pool:
            oh = (b_ref[...] ==
                  lax.broadcasted_iota(jnp.int32, (BM, NG), 1)
                  ).astype(jnp.float32)
            @pl.when(i == 0)
            def _():
                p_ref[...] = jnp.zeros_like(p_ref)
            p_ref[...] += lax.dot_general(
                oh, xp, (((0,), (0,)), ((), ())), precision=HI,
                preferred_element_type=jnp.float32)

    in_specs = [
        pl.BlockSpec((BM, dh), lambda i: (i, 0)),
        pl.BlockSpec((1, dh), lambda i: (0, 0)),
        pl.BlockSpec((1, dh), lambda i: (0, 0)),
        pl.BlockSpec((1, dh), lambda i: (0, 0)),
        pl.BlockSpec((1, dh), lambda i: (0, 0)),
    ]
    out_specs = [pl.BlockSpec((BM, dh), lambda i: (i, 0))]
    out_shape = [jax.ShapeDtypeStruct((np_, dh), jnp.float32)]
    args = [u, jnp.reshape(m, (1, -1)), jnp.reshape(v, (1, -1)),
            jnp.reshape(g, (1, -1)), jnp.reshape(be, (1, -1))]
    if with_pool:
        in_specs.append(pl.BlockSpec((BM, 1), lambda i: (i, 0)))
        out_specs.append(pl.BlockSpec((NG, dh), lambda i: (0, 0)))
        out_shape.append(jax.ShapeDtypeStruct((NG, dh), jnp.float32))
        args.append(jnp.reshape(pool_ids, (-1, 1)))
    res = pl.pallas_call(body, grid=grid, in_specs=in_specs,
                         out_specs=out_specs, out_shape=out_shape)(*args)
    return res if with_pool else res[0]


def _bn_inline(x, g, be):
    m = jnp.mean(x, axis=0, keepdims=True)
    d = x - m
    v = jnp.mean(d * d, axis=0, keepdims=True)
    return d / jnp.sqrt(v + 1e-5) * g + be


def _head(pool_a, pool_f, tgt, p):
    def body(pa_ref, pf_ref, t_ref, aw_ref, ab_ref, fw_ref, fb_ref,
             cw1_ref, cb1_ref, cg1_ref, ce1_ref, cw2_ref, cb2_ref,
             cg2_ref, ce2_ref, cfw_ref, cfb_ref,
             pw1_ref, pb1_ref, pg1_ref, pe1_ref, pw2_ref, pb2_ref,
             pw3_ref, pb3_ref,
             out_ref, a_ref, f_ref, c_ref):
        dot = functools.partial(jnp.dot, preferred_element_type=jnp.float32)
        a = jnp.maximum(dot(pa_ref[...], aw_ref[...]) + ab_ref[...], 0.0)
        f = jnp.maximum(dot(pf_ref[...], fw_ref[...]) + fb_ref[...], 0.0)
        c = jnp.maximum(dot(t_ref[...], cw1_ref[...]) + cb1_ref[...], 0.0)
        c = _bn_inline(c, cg1_ref[...], ce1_ref[...])
        c = jnp.maximum(dot(c, cw2_ref[...]) + cb2_ref[...], 0.0)
        c = _bn_inline(c, cg2_ref[...], ce2_ref[...])
        c = jnp.maximum(dot(c, cfw_ref[...]) + cfb_ref[...], 0.0)
        comb = jnp.concatenate([a, f, c], axis=1)
        o = jnp.maximum(dot(comb, pw1_ref[...]) + pb1_ref[...], 0.0)
        o = _bn_inline(o, pg1_ref[...], pe1_ref[...])
        o = jnp.maximum(dot(o, pw2_ref[...]) + pb2_ref[...], 0.0)
        o = dot(o, pw3_ref[...]) + pb3_ref[...]
        out_ref[...] = o
        a_ref[...] = a
        f_ref[...] = f
        c_ref[...] = c

    r = lambda a: jnp.reshape(a, (1, -1))
    w3 = jnp.pad(p['pred_W3'], ((0, 0), (0, 127)))
    b3 = jnp.pad(jnp.reshape(p['pred_b3'], (1, 1)), ((0, 0), (0, 127)))
    tgt_p = jnp.pad(tgt, ((0, 0), (0, 2)))
    cw1_p = jnp.pad(p['cell_W1'], ((0, 2), (0, 0)))
    out, a, f, c = pl.pallas_call(
        body,
        out_shape=[jax.ShapeDtypeStruct((NG, 128), jnp.float32),
                   jax.ShapeDtypeStruct((NG, 128), jnp.float32),
                   jax.ShapeDtypeStruct((NG, 128), jnp.float32),
                   jax.ShapeDtypeStruct((NG, 128), jnp.float32)],
    )(pool_a, pool_f, tgt_p,
      p['atom_fc_W'], r(p['atom_fc_b']), p['frag_fc_W'], r(p['frag_fc_b']),
      cw1_p, r(p['cell_b1']), r(p['cell_g1']), r(p['cell_be1']),
      p['cell_W2'], r(p['cell_b2']), r(p['cell_g2']), r(p['cell_be2']),
      p['cell_fc_W'], r(p['cell_fc_b']),
      p['pred_W1'], r(p['pred_b1']), r(p['pred_g1']), r(p['pred_be1']),
      p['pred_W2'], r(p['pred_b2']), w3, b3)
    return out[:, 0:1], a, f, c


def _gin_branch(x0, src, dst, pool_ids, layers, *, nreal, np_, ch, nchunk,
                nb):
    srcp = jnp.pad(src, (0, 16 * EB * nb - src.shape[0]))
    dstp = jnp.pad(dst, (0, 16 * EB * nb - dst.shape[0]),
                   constant_values=-1)
    poolp = _pad_rows(jnp.reshape(pool_ids, (-1, 1)), np_)
    h = x0
    pool = None
    for li, lp in enumerate(layers):
        d = h.shape[1]
        agg = _sc_edge_agg(h, srcp, dstp, np_=np_, ch=ch, nchunk=nchunk,
                           d=d, nb=nb)
        if lp['W1'].shape[0] != d:
            w1 = jnp.pad(lp['W1'], ((0, d - lp['W1'].shape[0]), (0, 0)))
        else:
            w1 = lp['W1']
        u, t = _gin_mlp(h, agg, lp['eps'], w1, lp['b1'], lp['W2'],
                        lp['b2'], nreal=nreal)
        us = jnp.dot(t, lp['W2']) + lp['b2']
        m = jnp.mean(us, axis=0)
        v = jnp.var(us, axis=0)
        last = li == len(layers) - 1
        if last:
            h, pool = _bn_relu(u, m, v, lp['g'], lp['be'], nreal=nreal,
                               pool_ids=poolp)
        else:
            h = _bn_relu(u, m, v, lp['g'], lp['be'], nreal=nreal)
    return pool


def kernel(x, edge_index, batch, frag, frag_edge_index, frag_batch, target,
           params):
    N, NF = x.shape[0], frag.shape[0]
    CH = 12544
    NP, NFP = 4 * CH, 2 * CH
    NBA = -(-edge_index.shape[1] // (16 * EB))
    NBF = -(-frag_edge_index.shape[1] // (16 * EB))
    p = params

    x_p = _pad_rows(jnp.pad(x, ((0, 0), (0, 50))), NP)
    pool_a = _gin_branch(x_p, edge_index[0], edge_index[1], batch,
                         p['atom'], nreal=N, np_=NP, ch=CH, nchunk=4,
                         nb=NBA)

    frag_p = _pad_rows(jnp.reshape(frag, (-1, 1)), NFP)[:, 0]
    xf0 = _sc_embed_gather(p['frag_emb'], frag_p, np_=NFP)
    pool_f = _gin_branch(xf0, frag_edge_index[0], frag_edge_index[1],
                         frag_batch, p['frag'], nreal=NF, np_=NFP, ch=CH,
                         nchunk=2, nb=NBF)

    return _head(pool_a, pool_f, target, p)
